# Initial kernel scaffold; baseline (speedup 1.0000x reference)
import jax
import jax.numpy as jnp
from jax.experimental import pallas as pl


def kernel(x, edge_index, batch, W1l, b1l, W1r, W2l, b2l, W2r, Wlin, blin):
    raise NotImplementedError("write your pallas kernel here")


# speedup vs baseline: 5.3493x; 5.3493x over previous
import functools

import jax
import jax.numpy as jnp
from jax import lax
from jax.experimental import pallas as pl
from jax.experimental.pallas import tpu as pltpu
from jax.experimental.pallas import tpu_sc as plsc

N = 10000
E = 320000
F_IN = 128
H = 128
OUT = 16
B = 64

NC = 2
NS = 16
NW = NC * NS
EP = E // NW
CH = 80
NCH = EP // CH
DEG_P = 10240
RPT = 624
TAIL = N - NS * RPT


def _make_sc_edge(with_deg):
  mesh = plsc.VectorSubcoreMesh(core_axis_name="c", subcore_axis_name="s")
  out_type = [jax.ShapeDtypeStruct((NC, N, H), jnp.float32)]
  scratch = [
      pltpu.VMEM((CH,), jnp.int32),
      pltpu.VMEM((CH,), jnp.int32),
      pltpu.VMEM((CH, H), jnp.float32),
      pltpu.VMEM_SHARED((N, H), jnp.float32),
      pltpu.SemaphoreType.DMA,
  ]
  if with_deg:
    out_type.append(jax.ShapeDtypeStruct((NC * N,), jnp.float32))
    scratch += [
        pltpu.VMEM((CH,), jnp.float32),
        pltpu.VMEM((208,), jnp.float32),
        pltpu.VMEM((RPT,), jnp.float32),
        pltpu.VMEM_SHARED((N,), jnp.float32),
    ]

  def body(y_hbm, src_hbm, dst_hbm, *rest):
    if with_deg:
      (acc_out, deg_out, sidx, didx, rows, acc_sh, sem,
       ones_v, zer_v, deg_buf, deg_sh) = rest
    else:
      acc_out, sidx, didx, rows, acc_sh, sem = rest
    c = lax.axis_index("c")
    s = lax.axis_index("s")
    wid = s * NC + c

    zero16 = jnp.zeros((16,), jnp.float32)

    def zrow(r, carry):
      for k in range(H // 16):
        rows[r, pl.ds(k * 16, 16)] = zero16
      return carry
    lax.fori_loop(0, CH, zrow, 0)

    base_r = s * RPT
    for k in range(RPT // CH):
      pltpu.sync_copy(rows, acc_sh.at[pl.ds(base_r + k * CH, CH)])
    rem = RPT % CH
    if rem:
      pltpu.sync_copy(rows.at[pl.ds(0, rem)],
                      acc_sh.at[pl.ds(base_r + (RPT // CH) * CH, rem)])

    @pl.when(s == 0)
    def _():
      pltpu.sync_copy(rows.at[pl.ds(0, TAIL)],
                      acc_sh.at[pl.ds(NS * RPT, TAIL)])

    if with_deg:
      ones16 = jnp.ones((16,), jnp.float32)
      for k in range(CH // 16):
        ones_v[pl.ds(k * 16, 16)] = ones16
      for k in range(208 // 16):
        zer_v[pl.ds(k * 16, 16)] = zero16
      for k in range(3):
        pltpu.sync_copy(zer_v, deg_sh.at[pl.ds(s * RPT + k * 208, 208)])

      @pl.when(s == 0)
      def _():
        pltpu.sync_copy(zer_v.at[pl.ds(0, TAIL)],
                        deg_sh.at[pl.ds(NS * RPT, TAIL)])

    plsc.subcore_barrier()

    ebase = wid * EP
    ones16 = jnp.ones((16,), jnp.float32)

    def edge_chunk(j, carry):
      off = pl.multiple_of(ebase + j * CH, 8)
      pltpu.sync_copy(src_hbm.at[pl.ds(off, CH)], sidx)
      pltpu.async_copy(y_hbm.at[sidx], rows, sem).wait()
      pltpu.sync_copy(dst_hbm.at[pl.ds(off, CH)], didx)
      pltpu.sync_copy(rows, acc_sh.at[didx], add=True)
      if with_deg:
        pltpu.sync_copy(ones_v, deg_sh.at[didx], add=True)
      return carry
    lax.fori_loop(0, NCH, edge_chunk, 0)

    plsc.subcore_barrier()
    if with_deg:
      pltpu.sync_copy(deg_sh.at[pl.ds(s * RPT, RPT)], deg_buf)
      pltpu.sync_copy(deg_buf, deg_out.at[pl.ds(c * N + s * RPT, RPT)])

      @pl.when(s == 0)
      def _():
        pltpu.sync_copy(deg_sh.at[pl.ds(NS * RPT, TAIL)],
                        deg_buf.at[pl.ds(0, TAIL)])
        pltpu.sync_copy(deg_buf.at[pl.ds(0, TAIL)],
                        deg_out.at[pl.ds(c * N + NS * RPT, TAIL)])

    pltpu.sync_copy(acc_sh.at[pl.ds(base_r, RPT)],
                    acc_out.at[c, pl.ds(base_r, RPT)])

    @pl.when(s == 0)
    def _():
      pltpu.sync_copy(acc_sh.at[pl.ds(NS * RPT, TAIL)],
                      acc_out.at[c, pl.ds(NS * RPT, TAIL)])

  return pl.kernel(body, out_type=out_type, mesh=mesh, scratch_types=scratch)


_sc_edge_deg = _make_sc_edge(True)
_sc_edge = _make_sc_edge(False)

BLK = 1000
NB = N // BLK


def _dot_t(a, w):
  return lax.dot_general(a, w, (((1,), (1,)), ((), ())),
                         preferred_element_type=jnp.float32)


def _degsum_body(dp_ref, out_ref):
  out_ref[...] = jnp.sum(dp_ref[...], axis=0, keepdims=True)


def _degsum(degp):
  return pl.pallas_call(
      _degsum_body,
      grid=(1,),
      in_specs=[pl.BlockSpec((NC, N), lambda i: (0, 0))],
      out_specs=pl.BlockSpec((1, N), lambda i: (0, 0)),
      out_shape=jax.ShapeDtypeStruct((1, N), jnp.float32),
  )(degp)


def _pre_body(x_ref, wl_ref, wr_ref, y_ref, r_ref):
  xb = x_ref[...]
  y_ref[...] = _dot_t(xb, wl_ref[...])
  r_ref[...] = _dot_t(xb, wr_ref[...])


def _pre(x, Wl, Wr):
  return pl.pallas_call(
      _pre_body,
      grid=(NB,),
      in_specs=[
          pl.BlockSpec((BLK, F_IN), lambda i: (i, 0)),
          pl.BlockSpec((H, F_IN), lambda i: (0, 0)),
          pl.BlockSpec((H, F_IN), lambda i: (0, 0)),
      ],
      out_specs=[pl.BlockSpec((BLK, H), lambda i: (i, 0))] * 2,
      out_shape=[jax.ShapeDtypeStruct((N, H), jnp.float32)] * 2,
  )(x, Wl, Wr)


def _mid_body(a0, a1, d, r1, bl, wl_ref, wr_ref, y_ref, r_ref):
  deg = jnp.maximum(d[...], 1.0)
  h = jnp.maximum((a0[...] + a1[...]) / deg + bl[...] + r1[...], 0.0)
  y_ref[...] = _dot_t(h, wl_ref[...])
  r_ref[...] = _dot_t(h, wr_ref[...])


def _mid(a0, a1, d, r1, bl, Wl, Wr):
  row = pl.BlockSpec((BLK, H), lambda i: (i, 0))
  col = pl.BlockSpec((BLK, 1), lambda i: (i, 0))
  return pl.pallas_call(
      _mid_body,
      grid=(NB,),
      in_specs=[row, row, col, row,
                pl.BlockSpec((1, H), lambda i: (0, 0)),
                pl.BlockSpec((H, H), lambda i: (0, 0)),
                pl.BlockSpec((H, H), lambda i: (0, 0))],
      out_specs=[row, row],
      out_shape=[jax.ShapeDtypeStruct((N, H), jnp.float32)] * 2,
  )(a0, a1, d, r1, bl, Wl, Wr)


def _post_body(batch_smem, a0, a1, d, r2, bl, bcol, wlin, blin,
               out_ref, pool):
  i = pl.program_id(0)
  deg = jnp.maximum(d[...], 1.0)
  h = jnp.maximum((a0[...] + a1[...]) / deg + bl[...] + r2[...], 0.0)

  @pl.when(i == 0)
  def _():
    pool[...] = jnp.full((B, H), -jnp.inf, jnp.float32)

  g_lo = batch_smem[i * BLK]
  g_hi = batch_smem[i * BLK + BLK - 1]
  bc = bcol[...]
  for g in range(B):
    @pl.when((g >= g_lo) & (g <= g_hi))
    def _():
      red = jnp.max(jnp.where(bc == g, h, -jnp.inf), axis=0, keepdims=True)
      pool[pl.ds(g, 1), :] = jnp.maximum(pool[pl.ds(g, 1), :], red)

  @pl.when(i == NB - 1)
  def _():
    out_ref[...] = _dot_t(pool[...], wlin[...]) + blin[...]


def _post(batch, a0, a1, d, r2, bl, bcol, Wlin, blin):
  row = pl.BlockSpec((BLK, H), lambda i, b_s: (i, 0))
  col = pl.BlockSpec((BLK, 1), lambda i, b_s: (i, 0))
  grid_spec = pltpu.PrefetchScalarGridSpec(
      num_scalar_prefetch=1,
      grid=(NB,),
      in_specs=[row, row, col, row,
                pl.BlockSpec((1, H), lambda i, b_s: (0, 0)),
                col,
                pl.BlockSpec((OUT, H), lambda i, b_s: (0, 0)),
                pl.BlockSpec((1, OUT), lambda i, b_s: (0, 0))],
      out_specs=pl.BlockSpec((B, OUT), lambda i, b_s: (0, 0)),
      scratch_shapes=[pltpu.VMEM((B, H), jnp.float32)],
  )
  return pl.pallas_call(
      _post_body,
      grid_spec=grid_spec,
      out_shape=jax.ShapeDtypeStruct((B, OUT), jnp.float32),
  )(batch, a0, a1, d, r2, bl, bcol, Wlin, blin)


def kernel(x, edge_index, batch, W1l, b1l, W1r, W2l, b2l, W2r, Wlin, blin):
  src = edge_index[0]
  dst = edge_index[1]

  y1, r1 = _pre(x, W1l, W1r)
  acc1, degp = _sc_edge_deg(y1, src, dst)
  deg = _degsum(degp.reshape(NC, N)).reshape(N, 1)
  y2, r2 = _mid(acc1[0], acc1[1], deg, r1, b1l.reshape(1, H), W2l, W2r)
  (acc2,) = _sc_edge(y2, src, dst)
  return _post(batch, acc2[0], acc2[1], deg, r2, b2l.reshape(1, H),
               batch.reshape(N, 1), Wlin, blin.reshape(1, OUT))


# speedup vs baseline: 12.8395x; 2.4002x over previous
import functools

import jax
import jax.numpy as jnp
from jax import lax
from jax.experimental import pallas as pl
from jax.experimental.pallas import tpu as pltpu
from jax.experimental.pallas import tpu_sc as plsc

N = 10000
E = 320000
F_IN = 128
H = 128
OUT = 16
B = 64

NC = 2
NS = 16
NW = NC * NS
EP = E // NW
CH = 40
NCH = EP // CH
DEG_P = 10240
RPT = 624
TAIL = N - NS * RPT


NBUF = 5


def _make_sc_edge(with_deg):
  mesh = plsc.VectorSubcoreMesh(core_axis_name="c", subcore_axis_name="s")
  out_type = [jax.ShapeDtypeStruct((NC, N, H), jnp.float32)]
  scratch = [
      pltpu.VMEM((EP,), jnp.int32),
      pltpu.VMEM((NBUF, CH), jnp.int32),
      pltpu.VMEM_SHARED((N, H), jnp.float32),
  ]
  scratch += [pltpu.VMEM((CH, H), jnp.float32) for _ in range(NBUF)]
  scratch += [pltpu.SemaphoreType.DMA for _ in range(2 * NBUF)]
  if with_deg:
    out_type.append(jax.ShapeDtypeStruct((NC * N,), jnp.float32))
    scratch += [
        pltpu.VMEM((CH,), jnp.float32),
        pltpu.VMEM((208,), jnp.float32),
        pltpu.VMEM((RPT,), jnp.float32),
        pltpu.VMEM_SHARED((N,), jnp.float32),
    ]

  def body(y_hbm, src_hbm, dst_hbm, *rest):
    if with_deg:
      (acc_out, deg_out, sidx_all, didx_ring, acc_sh, *bufs) = rest
      rows = bufs[:NBUF]
      sems = bufs[NBUF:2 * NBUF]
      dsems = bufs[2 * NBUF:3 * NBUF]
      ones_v, zer_v, deg_buf, deg_sh = bufs[3 * NBUF:]
    else:
      (acc_out, sidx_all, didx_ring, acc_sh, *bufs) = rest
      rows = bufs[:NBUF]
      sems = bufs[NBUF:2 * NBUF]
      dsems = bufs[2 * NBUF:3 * NBUF]
    c = lax.axis_index("c")
    s = lax.axis_index("s")
    wid = s * NC + c

    zero16 = jnp.zeros((16,), jnp.float32)

    def zrow(r, carry):
      for k in range(H // 16):
        rows[0][r, pl.ds(k * 16, 16)] = zero16
      return carry
    lax.fori_loop(0, CH, zrow, 0)

    pltpu.sync_copy(src_hbm.at[pl.ds(wid * EP, EP)], sidx_all)

    base_r = s * RPT
    for k in range(RPT // CH):
      pltpu.sync_copy(rows[0], acc_sh.at[pl.ds(base_r + k * CH, CH)])
    rem = RPT % CH
    if rem:
      pltpu.sync_copy(rows[0].at[pl.ds(0, rem)],
                      acc_sh.at[pl.ds(base_r + (RPT // CH) * CH, rem)])

    @pl.when(s == 0)
    def _():
      pltpu.sync_copy(rows[0].at[pl.ds(0, TAIL)],
                      acc_sh.at[pl.ds(NS * RPT, TAIL)])

    if with_deg:
      ones16 = jnp.ones((16,), jnp.float32)
      ones_v[pl.ds(0, 16)] = ones16
      ones_v[pl.ds(16, 16)] = ones16
      ones_v[pl.ds(CH - 16, 16)] = ones16
      for k in range(208 // 16):
        zer_v[pl.ds(k * 16, 16)] = zero16
      for k in range(3):
        pltpu.sync_copy(zer_v, deg_sh.at[pl.ds(s * RPT + k * 208, 208)])

      @pl.when(s == 0)
      def _():
        pltpu.sync_copy(zer_v.at[pl.ds(0, TAIL)],
                        deg_sh.at[pl.ds(NS * RPT, TAIL)])

    plsc.subcore_barrier()

    def start_fetch(j, b):
      off = pl.multiple_of(j * CH, 8)
      pltpu.async_copy(y_hbm.at[sidx_all.at[pl.ds(off, CH)]],
                       rows[b], sems[b])
      pltpu.async_copy(dst_hbm.at[wid, j, 0], didx_ring.at[b], dsems[b])

    for b in range(NBUF):
      start_fetch(b, b)

    @pl.loop(0, NCH, step=NBUF)
    def _(j0):
      for b in range(NBUF):
        j = j0 + b
        pltpu.make_async_copy(y_hbm.at[sidx_all.at[pl.ds(0, CH)]],
                              rows[b], sems[b]).wait()
        pltpu.make_async_copy(dst_hbm.at[wid, 0, 0], didx_ring.at[b],
                              dsems[b]).wait()
        dj = didx_ring.at[b]
        pltpu.sync_copy(rows[b], acc_sh.at[dj], add=True)
        if with_deg:
          pltpu.sync_copy(ones_v, deg_sh.at[dj], add=True)

        @pl.when(j + NBUF < NCH)
        def _():
          start_fetch(j + NBUF, b)

    plsc.subcore_barrier()
    if with_deg:
      pltpu.sync_copy(deg_sh.at[pl.ds(s * RPT, RPT)], deg_buf)
      pltpu.sync_copy(deg_buf, deg_out.at[pl.ds(c * N + s * RPT, RPT)])

      @pl.when(s == 0)
      def _():
        pltpu.sync_copy(deg_sh.at[pl.ds(NS * RPT, TAIL)],
                        deg_buf.at[pl.ds(0, TAIL)])
        pltpu.sync_copy(deg_buf.at[pl.ds(0, TAIL)],
                        deg_out.at[pl.ds(c * N + NS * RPT, TAIL)])

    pltpu.sync_copy(acc_sh.at[pl.ds(base_r, RPT)],
                    acc_out.at[c, pl.ds(base_r, RPT)])

    @pl.when(s == 0)
    def _():
      pltpu.sync_copy(acc_sh.at[pl.ds(NS * RPT, TAIL)],
                      acc_out.at[c, pl.ds(NS * RPT, TAIL)])

  return pl.kernel(body, out_type=out_type, mesh=mesh, scratch_types=scratch)


_sc_edge_deg = _make_sc_edge(True)
_sc_edge = _make_sc_edge(False)

BLK = 1000
NB = N // BLK


def _dot_t(a, w):
  return lax.dot_general(a, w, (((1,), (1,)), ((), ())),
                         preferred_element_type=jnp.float32)


def _degsum_body(dp_ref, out_ref):
  out_ref[...] = jnp.sum(dp_ref[...], axis=0, keepdims=True)


def _degsum(degp):
  return pl.pallas_call(
      _degsum_body,
      grid=(1,),
      in_specs=[pl.BlockSpec((NC, N), lambda i: (0, 0))],
      out_specs=pl.BlockSpec((1, N), lambda i: (0, 0)),
      out_shape=jax.ShapeDtypeStruct((1, N), jnp.float32),
  )(degp)


def _pre_body(x_ref, wl_ref, wr_ref, y_ref, r_ref):
  xb = x_ref[...]
  y_ref[...] = _dot_t(xb, wl_ref[...])
  r_ref[...] = _dot_t(xb, wr_ref[...])


def _pre(x, Wl, Wr):
  return pl.pallas_call(
      _pre_body,
      grid=(NB,),
      in_specs=[
          pl.BlockSpec((BLK, F_IN), lambda i: (i, 0)),
          pl.BlockSpec((H, F_IN), lambda i: (0, 0)),
          pl.BlockSpec((H, F_IN), lambda i: (0, 0)),
      ],
      out_specs=[pl.BlockSpec((BLK, H), lambda i: (i, 0))] * 2,
      out_shape=[jax.ShapeDtypeStruct((N, H), jnp.float32)] * 2,
  )(x, Wl, Wr)


def _mid_body(a0, a1, d, r1, bl, wl_ref, wr_ref, y_ref, r_ref):
  deg = jnp.maximum(d[...], 1.0)
  h = jnp.maximum((a0[...] + a1[...]) / deg + bl[...] + r1[...], 0.0)
  y_ref[...] = _dot_t(h, wl_ref[...])
  r_ref[...] = _dot_t(h, wr_ref[...])


def _mid(a0, a1, d, r1, bl, Wl, Wr):
  row = pl.BlockSpec((BLK, H), lambda i: (i, 0))
  col = pl.BlockSpec((BLK, 1), lambda i: (i, 0))
  return pl.pallas_call(
      _mid_body,
      grid=(NB,),
      in_specs=[row, row, col, row,
                pl.BlockSpec((1, H), lambda i: (0, 0)),
                pl.BlockSpec((H, H), lambda i: (0, 0)),
                pl.BlockSpec((H, H), lambda i: (0, 0))],
      out_specs=[row, row],
      out_shape=[jax.ShapeDtypeStruct((N, H), jnp.float32)] * 2,
  )(a0, a1, d, r1, bl, Wl, Wr)


def _post_body(batch_smem, a0, a1, d, r2, bl, bcol, wlin, blin,
               out_ref, pool):
  i = pl.program_id(0)
  deg = jnp.maximum(d[...], 1.0)
  h = jnp.maximum((a0[...] + a1[...]) / deg + bl[...] + r2[...], 0.0)

  @pl.when(i == 0)
  def _():
    pool[...] = jnp.full((B, H), -jnp.inf, jnp.float32)

  g_lo = batch_smem[i * BLK]
  g_hi = batch_smem[i * BLK + BLK - 1]
  bc = bcol[...]
  for g in range(B):
    @pl.when((g >= g_lo) & (g <= g_hi))
    def _():
      red = jnp.max(jnp.where(bc == g, h, -jnp.inf), axis=0, keepdims=True)
      pool[pl.ds(g, 1), :] = jnp.maximum(pool[pl.ds(g, 1), :], red)

  @pl.when(i == NB - 1)
  def _():
    out_ref[...] = _dot_t(pool[...], wlin[...]) + blin[...]


def _post(batch, a0, a1, d, r2, bl, bcol, Wlin, blin):
  row = pl.BlockSpec((BLK, H), lambda i, b_s: (i, 0))
  col = pl.BlockSpec((BLK, 1), lambda i, b_s: (i, 0))
  grid_spec = pltpu.PrefetchScalarGridSpec(
      num_scalar_prefetch=1,
      grid=(NB,),
      in_specs=[row, row, col, row,
                pl.BlockSpec((1, H), lambda i, b_s: (0, 0)),
                col,
                pl.BlockSpec((OUT, H), lambda i, b_s: (0, 0)),
                pl.BlockSpec((1, OUT), lambda i, b_s: (0, 0))],
      out_specs=pl.BlockSpec((B, OUT), lambda i, b_s: (0, 0)),
      scratch_shapes=[pltpu.VMEM((B, H), jnp.float32)],
  )
  return pl.pallas_call(
      _post_body,
      grid_spec=grid_spec,
      out_shape=jax.ShapeDtypeStruct((B, OUT), jnp.float32),
  )(batch, a0, a1, d, r2, bl, bcol, Wlin, blin)


def kernel(x, edge_index, batch, W1l, b1l, W1r, W2l, b2l, W2r, Wlin, blin):
  src = edge_index[0]
  dst = edge_index[1]

  dst = dst.reshape(NW, NCH, 1, CH)
  y1, r1 = _pre(x, W1l, W1r)
  acc1, degp = _sc_edge_deg(y1, src, dst)
  deg = _degsum(degp.reshape(NC, N)).reshape(N, 1)
  y2, r2 = _mid(acc1[0], acc1[1], deg, r1, b1l.reshape(1, H), W2l, W2r)
  (acc2,) = _sc_edge(y2, src, dst)
  return _post(batch, acc2[0], acc2[1], deg, r2, b2l.reshape(1, H),
               batch.reshape(N, 1), Wlin, blin.reshape(1, OUT))
